# Initial kernel scaffold; baseline (speedup 1.0000x reference)
import jax
import jax.numpy as jnp
from jax.experimental import pallas as pl


def kernel(x, edge_index, W1, b1, W2, b2):
    raise NotImplementedError("write your pallas kernel here")


# speedup vs baseline: 2.2684x; 2.2684x over previous
import functools

import jax
import jax.numpy as jnp
from jax import lax
from jax.experimental import pallas as pl

N = 50000
D_IN = 1433
HID = 256
D_OUT = 7
MIN_T = 1.0 + 1e-6

_INTERPRET = False

R_B = 1000
R_D = 2000
R_F = 2000


def _acosh(t):
    return jnp.log(t + jnp.sqrt(t * t - 1.0))


def _dinv(deg):
    return lax.rsqrt(jnp.maximum(deg, 1.0))


def _phase_b_body(x_ref, w1t_ref, b1_ref, w1c0_ref, mask_ref, deg_ref, out_ref):
    xs = x_ref[...]
    x0 = xs[:, 0:1]
    rowsq = jnp.sum(xs * xs, axis=1, keepdims=True)
    yn2 = rowsq - x0 * x0
    theta = jnp.maximum(x0, MIN_T)
    scale = _acosh(theta) / jnp.sqrt(yn2 + 1e-15)
    mm = jnp.dot(xs, w1t_ref[...], preferred_element_type=jnp.float32)
    h1 = scale * (mm - x0 * w1c0_ref[...]) + b1_ref[...]
    t1 = h1 * mask_ref[...] * _dinv(deg_ref[...])
    out_ref[...] = t1


def _phase_b(x, w1t, b1r, w1c0, mask, deg):
    grid = (N // R_B,)
    return pl.pallas_call(
        _phase_b_body,
        grid=grid,
        in_specs=[
            pl.BlockSpec((R_B, D_IN), lambda i: (i, 0)),
            pl.BlockSpec((D_IN, HID), lambda i: (0, 0)),
            pl.BlockSpec((1, HID), lambda i: (0, 0)),
            pl.BlockSpec((1, HID), lambda i: (0, 0)),
            pl.BlockSpec((1, HID), lambda i: (0, 0)),
            pl.BlockSpec((R_B, 1), lambda i: (i, 0)),
        ],
        out_specs=pl.BlockSpec((R_B, HID), lambda i: (i, 0)),
        out_shape=jax.ShapeDtypeStruct((N, HID), jnp.float32),
        interpret=_INTERPRET,
    )(x, w1t, b1r, w1c0, mask, deg)


def _phase_d_body(agg_ref, w2t_ref, b2_ref, mask_ref, deg_ref, out_ref):
    dinv = _dinv(deg_ref[...])
    u2 = jax.nn.relu(agg_ref[...] * dinv)
    h2 = jnp.dot(u2, w2t_ref[...], preferred_element_type=jnp.float32)
    out_ref[...] = (h2 + b2_ref[...]) * mask_ref[...] * dinv


def _phase_d(agg, w2t, b2r, mask, deg):
    grid = (N // R_D,)
    return pl.pallas_call(
        _phase_d_body,
        grid=grid,
        in_specs=[
            pl.BlockSpec((R_D, HID), lambda i: (i, 0)),
            pl.BlockSpec((HID, 16), lambda i: (0, 0)),
            pl.BlockSpec((1, 16), lambda i: (0, 0)),
            pl.BlockSpec((1, 16), lambda i: (0, 0)),
            pl.BlockSpec((R_D, 1), lambda i: (i, 0)),
        ],
        out_specs=pl.BlockSpec((R_D, 16), lambda i: (i, 0)),
        out_shape=jax.ShapeDtypeStruct((N, 16), jnp.float32),
        interpret=_INTERPRET,
    )(agg, w2t, b2r, mask, deg)


def _phase_f_body(agg2_ref, deg_ref, out_ref):
    a = agg2_ref[...] * _dinv(deg_ref[...])
    yn2 = jnp.sum(a * a, axis=1, keepdims=True)
    yn = jnp.sqrt(yn2 + 1e-15)
    s = 0.5 * (jnp.exp(yn) - jnp.exp(-yn)) / yn
    xr = a * s
    o0 = jnp.sqrt(1.0 + jnp.sum(xr * xr, axis=1, keepdims=True))
    out_ref[...] = jnp.concatenate([o0, xr[:, 1:D_OUT]], axis=1)


def _phase_f(agg2, deg):
    grid = (N // R_F,)
    return pl.pallas_call(
        _phase_f_body,
        grid=grid,
        in_specs=[
            pl.BlockSpec((R_F, 16), lambda i: (i, 0)),
            pl.BlockSpec((R_F, 1), lambda i: (i, 0)),
        ],
        out_specs=pl.BlockSpec((R_F, D_OUT), lambda i: (i, 0)),
        out_shape=jax.ShapeDtypeStruct((N, D_OUT), jnp.float32),
        interpret=_INTERPRET,
    )(agg2, deg)


def kernel(x, edge_index, W1, b1, W2, b2):
    src = edge_index[0]
    dst = edge_index[1]

    deg = jnp.zeros((N,), jnp.float32).at[dst].add(1.0).reshape(N, 1)

    w1t = W1.T
    b1r = b1.reshape(1, HID)
    w1c0 = W1[:, 0].reshape(1, HID)
    mask_h = jnp.ones((1, HID), jnp.float32).at[0, 0].set(0.0)

    t1 = _phase_b(x, w1t, b1r, w1c0, mask_h, deg)

    agg1 = jnp.zeros((N, HID), jnp.float32).at[dst].add(t1[src])

    w2t = jnp.zeros((HID, 16), jnp.float32).at[:, :D_OUT].set(W2.T)
    b2r = jnp.zeros((1, 16), jnp.float32).at[0, :D_OUT].set(b2)
    mask16 = jnp.zeros((1, 16), jnp.float32).at[0, 1:D_OUT].set(1.0)

    t2 = _phase_d(agg1, w2t, b2r, mask16, deg)

    agg2 = jnp.zeros((N, 16), jnp.float32).at[dst].add(t2[src])

    return _phase_f(agg2, deg)


# speedup vs baseline: 3.8123x; 1.6806x over previous
import functools

import jax
import jax.numpy as jnp
from jax import lax
from jax.experimental import pallas as pl
from jax.experimental.pallas import tpu as pltpu
from jax.experimental.pallas import tpu_sc as plsc

N = 50000
D_IN = 1433
HID = 256
D_OUT = 7
MIN_T = 1.0 + 1e-6
E = 800000

NC, NS = 2, 16
CHUNK = 128
E_PAD = 819200
ROWS_ALL = E_PAD // CHUNK
NP = 50048
SINK = 50040
HALF = 25000
CBUF = 25024
CSINK = 25000
FK = 4
FW = 64

_INTERPRET = False

R_B = 1000
R_D = 2000
R_F = 2000


def _acosh(t):
    return jnp.log(t + jnp.sqrt(t * t - 1.0))


def _dinv2(d0, d1):
    return lax.rsqrt(jnp.maximum(d0 + d1, 1.0))


def _deg_body(dstr, zeros, out, dst_v, ones_v, zbuf_v, deg_sh):
    c = lax.axis_index("c")
    s = lax.axis_index("s")
    wid = c * NS + s
    for i in range(CHUNK // 16):
        ones_v[pl.ds(i * 16, 16)] = jnp.ones((16,), jnp.float32)
    pltpu.sync_copy(zeros.at[pl.ds(0, 3128)], zbuf_v)
    pltpu.sync_copy(zbuf_v, deg_sh.at[pl.ds(s * 3128, 3128)])
    plsc.subcore_barrier()
    nrows = ROWS_ALL // (NC * NS)
    pltpu.sync_copy(dstr.at[pl.ds(wid * nrows, nrows)], dst_v)

    def step(j, carry):
        pltpu.sync_copy(ones_v, deg_sh.at[dst_v.at[j]], add=True)
        return carry

    lax.fori_loop(0, nrows, step, 0)
    plsc.subcore_barrier()
    pltpu.sync_copy(deg_sh.at[pl.ds(s * 3128, 3128)], zbuf_v)
    pltpu.sync_copy(zbuf_v, out.at[pl.ds(c * NP + s * 3128, 3128)])


def _deg_sc(dst_r, zeros_np):
    nrows = ROWS_ALL // (NC * NS)
    return pl.kernel(
        _deg_body,
        out_type=jax.ShapeDtypeStruct((NC * NP,), jnp.float32),
        mesh=plsc.VectorSubcoreMesh(core_axis_name="c", subcore_axis_name="s"),
        scratch_types=[
            pltpu.VMEM((nrows, CHUNK), jnp.int32),
            pltpu.VMEM((CHUNK,), jnp.float32),
            pltpu.VMEM((3128,), jnp.float32),
            pltpu.VMEM_SHARED((NP,), jnp.float32),
        ],
        compiler_params=pltpu.CompilerParams(use_tc_tiling_on_sc=False),
        interpret=_INTERPRET,
    )(dst_r, zeros_np)


GB = 8


def _aggw_body(t0, t1, t2, t3, srcr, dstr, zeros, o0, o1, o2, o3,
               sst_v, dst_v, rows_v, zbuf_v, buf_sh):
    c = lax.axis_index("c")
    s = lax.axis_index("s")
    lo = c * HALF
    nrows = ROWS_ALL // NS
    r0 = s * nrows

    zrows = 1568
    zoff = jnp.minimum(s * zrows, CBUF - zrows)
    ooff = jnp.minimum(s * zrows, HALF - zrows)
    for tp, op in zip((t0, t1, t2, t3), (o0, o1, o2, o3)):
        pltpu.sync_copy(zeros.at[pl.ds(0, 56)], zbuf_v)
        for i in range(28):
            pltpu.sync_copy(zbuf_v, buf_sh.at[pl.ds(zoff + i * 56, 56)])
        plsc.subcore_barrier()

        def group(g, carry):
            pltpu.sync_copy(srcr.at[pl.ds(r0 + g * GB, GB)], sst_v)
            pltpu.sync_copy(dstr.at[pl.ds(r0 + g * GB, GB)], dst_v)
            for j in range(GB):
                for k in range(CHUNK // 16):
                    d = dst_v[j, pl.ds(k * 16, 16)]
                    hit = (d >= lo) & (d < lo + HALF)
                    dst_v[j, pl.ds(k * 16, 16)] = jnp.where(hit, d - lo, CSINK)
            for j in range(GB):
                pltpu.sync_copy(tp.at[sst_v.at[j]], rows_v)
                pltpu.sync_copy(rows_v, buf_sh.at[dst_v.at[j]], add=True)
            return carry

        lax.fori_loop(0, nrows // GB, group, 0)
        plsc.subcore_barrier()
        for i in range(28):
            pltpu.sync_copy(buf_sh.at[pl.ds(ooff + i * 56, 56)], zbuf_v)
            pltpu.sync_copy(zbuf_v, op.at[pl.ds(lo + ooff + i * 56, 56)])
        plsc.subcore_barrier()


def _aggw_sc(tchunks, src_r, dst_r, zeros_c):
    return pl.kernel(
        _aggw_body,
        out_type=[jax.ShapeDtypeStruct((N, FW), jnp.float32)] * FK,
        mesh=plsc.VectorSubcoreMesh(core_axis_name="c", subcore_axis_name="s"),
        scratch_types=[
            pltpu.VMEM((GB, CHUNK), jnp.int32),
            pltpu.VMEM((GB, CHUNK), jnp.int32),
            pltpu.VMEM((CHUNK, FW), jnp.float32),
            pltpu.VMEM((56, FW), jnp.float32),
            pltpu.VMEM_SHARED((CBUF, FW), jnp.float32),
        ],
        compiler_params=pltpu.CompilerParams(use_tc_tiling_on_sc=False),
        interpret=_INTERPRET,
    )(*tchunks, src_r, dst_r, zeros_c)


def _aggn_body(t2t, srcr, dstr, zeros, out, src_v, dst_v, rows_v, zbuf_v,
               agg_sh):
    c = lax.axis_index("c")
    s = lax.axis_index("s")
    wid = c * NS + s
    pltpu.sync_copy(zeros.at[pl.ds(0, 391)], zbuf_v)
    for i in range(8):
        pltpu.sync_copy(zbuf_v, agg_sh.at[pl.ds(s * 3128 + i * 391, 391)])
    plsc.subcore_barrier()
    nrows = ROWS_ALL // (NC * NS)
    pltpu.sync_copy(srcr.at[pl.ds(wid * nrows, nrows)], src_v)
    pltpu.sync_copy(dstr.at[pl.ds(wid * nrows, nrows)], dst_v)

    def step(j, carry):
        pltpu.sync_copy(t2t.at[src_v.at[j]], rows_v)
        pltpu.sync_copy(rows_v, agg_sh.at[dst_v.at[j]], add=True)
        return carry

    lax.fori_loop(0, nrows, step, 0)
    plsc.subcore_barrier()
    for i in range(8):
        pltpu.sync_copy(agg_sh.at[pl.ds(s * 3128 + i * 391, 391)], zbuf_v)
        pltpu.sync_copy(zbuf_v, out.at[pl.ds(c * NP + s * 3128 + i * 391, 391)])


def _aggn_sc(t2, src_r, dst_r, zeros_e):
    nrows = ROWS_ALL // (NC * NS)
    return pl.kernel(
        _aggn_body,
        out_type=jax.ShapeDtypeStruct((NC * NP, 16), jnp.float32),
        mesh=plsc.VectorSubcoreMesh(core_axis_name="c", subcore_axis_name="s"),
        scratch_types=[
            pltpu.VMEM((nrows, CHUNK), jnp.int32),
            pltpu.VMEM((nrows, CHUNK), jnp.int32),
            pltpu.VMEM((CHUNK, 16), jnp.float32),
            pltpu.VMEM((391, 16), jnp.float32),
            pltpu.VMEM_SHARED((NP, 16), jnp.float32),
        ],
        compiler_params=pltpu.CompilerParams(use_tc_tiling_on_sc=False),
        interpret=_INTERPRET,
    )(t2, src_r, dst_r, zeros_e)


def _phase_b_body(x_ref, w1t_ref, b1_ref, w1c0_ref, mask_ref, d0_ref, d1_ref,
                  *out_refs):
    xs = x_ref[...]
    x0 = xs[:, 0:1]
    rowsq = jnp.sum(xs * xs, axis=1, keepdims=True)
    yn2 = rowsq - x0 * x0
    theta = jnp.maximum(x0, MIN_T)
    scale = _acosh(theta) / jnp.sqrt(yn2 + 1e-15)
    mm = jnp.dot(xs, w1t_ref[...], preferred_element_type=jnp.float32)
    h1 = scale * (mm - x0 * w1c0_ref[...]) + b1_ref[...]
    t1 = h1 * mask_ref[...] * _dinv2(d0_ref[...], d1_ref[...])
    for p, oref in enumerate(out_refs):
        oref[...] = t1[:, p * FW:(p + 1) * FW]


def _phase_b(x, w1t, b1r, w1c0, mask, d0, d1):
    grid = (N // R_B,)
    return pl.pallas_call(
        _phase_b_body,
        grid=grid,
        in_specs=[
            pl.BlockSpec((R_B, D_IN), lambda i: (i, 0)),
            pl.BlockSpec((D_IN, HID), lambda i: (0, 0)),
            pl.BlockSpec((1, HID), lambda i: (0, 0)),
            pl.BlockSpec((1, HID), lambda i: (0, 0)),
            pl.BlockSpec((1, HID), lambda i: (0, 0)),
            pl.BlockSpec((R_B, 1), lambda i: (i, 0)),
            pl.BlockSpec((R_B, 1), lambda i: (i, 0)),
        ],
        out_specs=[pl.BlockSpec((R_B, FW), lambda i: (i, 0))] * FK,
        out_shape=[jax.ShapeDtypeStruct((N, FW), jnp.float32)] * FK,
        interpret=_INTERPRET,
    )(x, w1t, b1r, w1c0, mask, d0, d1)


def _phase_d_body(a0_ref, a1_ref, a2_ref, a3_ref, w2t_ref, b2_ref, mask_ref,
                  d0_ref, d1_ref, out_ref):
    agg = jnp.concatenate(
        [a0_ref[...], a1_ref[...], a2_ref[...], a3_ref[...]], axis=1)
    dinv = _dinv2(d0_ref[...], d1_ref[...])
    u2 = jax.nn.relu(agg * dinv)
    h2 = jnp.dot(u2, w2t_ref[...], preferred_element_type=jnp.float32)
    out_ref[...] = (h2 + b2_ref[...]) * mask_ref[...] * dinv


def _phase_d(aggs, w2t, b2r, mask, d0, d1):
    grid = (N // R_D,)
    return pl.pallas_call(
        _phase_d_body,
        grid=grid,
        in_specs=[pl.BlockSpec((R_D, FW), lambda i: (i, 0))] * FK + [
            pl.BlockSpec((HID, 16), lambda i: (0, 0)),
            pl.BlockSpec((1, 16), lambda i: (0, 0)),
            pl.BlockSpec((1, 16), lambda i: (0, 0)),
            pl.BlockSpec((R_D, 1), lambda i: (i, 0)),
            pl.BlockSpec((R_D, 1), lambda i: (i, 0)),
        ],
        out_specs=pl.BlockSpec((R_D, 16), lambda i: (i, 0)),
        out_shape=jax.ShapeDtypeStruct((N, 16), jnp.float32),
        interpret=_INTERPRET,
    )(*aggs, w2t, b2r, mask, d0, d1)


def _phase_f_body(p0_ref, p1_ref, d0_ref, d1_ref, out_ref):
    a = (p0_ref[...] + p1_ref[...]) * _dinv2(d0_ref[...], d1_ref[...])
    yn2 = jnp.sum(a * a, axis=1, keepdims=True)
    yn = jnp.sqrt(yn2 + 1e-15)
    s = 0.5 * (jnp.exp(yn) - jnp.exp(-yn)) / yn
    xr = a * s
    o0 = jnp.sqrt(1.0 + jnp.sum(xr * xr, axis=1, keepdims=True))
    out_ref[...] = jnp.concatenate([o0, xr[:, 1:D_OUT]], axis=1)


def _phase_f(p0, p1, d0, d1):
    grid = (N // R_F,)
    return pl.pallas_call(
        _phase_f_body,
        grid=grid,
        in_specs=[
            pl.BlockSpec((R_F, 16), lambda i: (i, 0)),
            pl.BlockSpec((R_F, 16), lambda i: (i, 0)),
            pl.BlockSpec((R_F, 1), lambda i: (i, 0)),
            pl.BlockSpec((R_F, 1), lambda i: (i, 0)),
        ],
        out_specs=pl.BlockSpec((R_F, D_OUT), lambda i: (i, 0)),
        out_shape=jax.ShapeDtypeStruct((N, D_OUT), jnp.float32),
        interpret=_INTERPRET,
    )(p0, p1, d0, d1)


def kernel(x, edge_index, W1, b1, W2, b2):
    src = edge_index[0].astype(jnp.int32)
    dst = edge_index[1].astype(jnp.int32)
    src_r = jnp.concatenate(
        [src, jnp.zeros((E_PAD - E,), jnp.int32)]).reshape(ROWS_ALL, CHUNK)
    dst_r = jnp.concatenate(
        [dst, jnp.full((E_PAD - E,), SINK, jnp.int32)]).reshape(ROWS_ALL, CHUNK)

    zeros_np = jnp.zeros((NP,), jnp.float32)
    zeros_c = jnp.zeros((CBUF, FW), jnp.float32)
    zeros_e = jnp.zeros((NP, 16), jnp.float32)

    deg2 = _deg_sc(dst_r, zeros_np).reshape(NC, NP)
    d0 = deg2[0].reshape(NP, 1)
    d1 = deg2[1].reshape(NP, 1)

    w1t = W1.T
    b1r = b1.reshape(1, HID)
    w1c0 = W1[:, 0].reshape(1, HID)
    mask_h = jnp.ones((1, HID), jnp.float32).at[0, 0].set(0.0)

    tchunks = _phase_b(x, w1t, b1r, w1c0, mask_h, d0[:N], d1[:N])

    aggs = _aggw_sc(tchunks, src_r, dst_r, zeros_c)

    w2t = jnp.zeros((HID, 16), jnp.float32).at[:, :D_OUT].set(W2.T)
    b2r = jnp.zeros((1, 16), jnp.float32).at[0, :D_OUT].set(b2)
    mask16 = jnp.zeros((1, 16), jnp.float32).at[0, 1:D_OUT].set(1.0)

    t2 = _phase_d(aggs, w2t, b2r, mask16, d0[:N], d1[:N])

    agg2 = _aggn_sc(t2, src_r, dst_r, zeros_e).reshape(NC, NP, 16)

    return _phase_f(agg2[0], agg2[1], d0, d1)


# speedup vs baseline: 4.1781x; 1.0960x over previous
import functools

import jax
import jax.numpy as jnp
from jax import lax
from jax.experimental import pallas as pl
from jax.experimental.pallas import tpu as pltpu
from jax.experimental.pallas import tpu_sc as plsc

N = 50000
D_IN = 1433
HID = 256
D_OUT = 7
MIN_T = 1.0 + 1e-6
E = 800000

NC, NS = 2, 16
CHUNK = 128
E_PAD = 819200
ROWS_ALL = E_PAD // CHUNK
NP = 50048
SINK = 50040
HALF = 25000
CBUF = 25024
CSINK = 25000
FK = 4
FW = 64

_INTERPRET = False

R_B = 1000
R_D = 2000
R_F = 2000


def _acosh(t):
    return jnp.log(t + jnp.sqrt(t * t - 1.0))


def _dinv2(d0, d1):
    return lax.rsqrt(jnp.maximum(d0 + d1, 1.0))


def _deg_body(dstr, zeros, out, dst_v, ones_v, zbuf_v, deg_sh):
    c = lax.axis_index("c")
    s = lax.axis_index("s")
    wid = c * NS + s
    for i in range(CHUNK // 16):
        ones_v[pl.ds(i * 16, 16)] = jnp.ones((16,), jnp.float32)
    pltpu.sync_copy(zeros.at[pl.ds(0, 3128)], zbuf_v)
    pltpu.sync_copy(zbuf_v, deg_sh.at[pl.ds(s * 3128, 3128)])
    plsc.subcore_barrier()
    nrows = ROWS_ALL // (NC * NS)
    pltpu.sync_copy(dstr.at[pl.ds(wid * nrows, nrows)], dst_v)

    def step(j, carry):
        pltpu.sync_copy(ones_v, deg_sh.at[dst_v.at[j]], add=True)
        return carry

    lax.fori_loop(0, nrows, step, 0)
    plsc.subcore_barrier()
    pltpu.sync_copy(deg_sh.at[pl.ds(s * 3128, 3128)], zbuf_v)
    pltpu.sync_copy(zbuf_v, out.at[pl.ds(c * NP + s * 3128, 3128)])


def _deg_sc(dst_r, zeros_np):
    nrows = ROWS_ALL // (NC * NS)
    return pl.kernel(
        _deg_body,
        out_type=jax.ShapeDtypeStruct((NC * NP,), jnp.float32),
        mesh=plsc.VectorSubcoreMesh(core_axis_name="c", subcore_axis_name="s"),
        scratch_types=[
            pltpu.VMEM((nrows, CHUNK), jnp.int32),
            pltpu.VMEM((CHUNK,), jnp.float32),
            pltpu.VMEM((3128,), jnp.float32),
            pltpu.VMEM_SHARED((NP,), jnp.float32),
        ],
        compiler_params=pltpu.CompilerParams(use_tc_tiling_on_sc=False, needs_layout_passes=False),
        interpret=_INTERPRET,
    )(dst_r, zeros_np)


SGR = 16
CCAP = SGR * CHUNK + CHUNK


ROWW = CHUNK * FW


SGRC = SGR * CHUNK


def _aggw_body(t0, t1, t2, t3, srcf, dstf, zeros, o0, o1, o2, o3,
               sst_v, dst_v, csrc_v, cdst_v, sidx0, sidx1, didx0, didx1,
               rows0, rows1, gs0, gs1, as0, as1, ss0, ss1, zs, os0, os1,
               buf_sh):
    c = lax.axis_index("c")
    s = lax.axis_index("s")
    lo = c * HALF
    hi = lo + HALF
    nrows = ROWS_ALL // NS
    e0 = s * nrows * CHUNK
    ngr = nrows // SGR

    def extract0(v):
        return lax.squeeze(lax.slice(v, (0,), (1,)), (0,))

    def drain(dummy_hbm, buf, sem):
        pltpu.make_async_copy(dummy_hbm, buf, sem).wait()

    zrows = 1568
    zoff = jnp.minimum(s * zrows, CBUF - zrows)
    ooff = jnp.minimum(s * zrows, HALF - zrows)
    for tp, op in zip((t0, t1, t2, t3), (o0, o1, o2, o3)):
        rows = (rows0, rows1)
        sidx = (sidx0, sidx1)
        didx = (didx0, didx1)
        gsem = (gs0, gs1)
        asem = (as0, as1)
        osem = (os0, os1)

        pltpu.async_copy(zeros.at[pl.ds(0, CHUNK)], rows0, gs0).wait()
        zds = [pltpu.async_copy(rows0.at[pl.ds(0, 112)],
                                buf_sh.at[pl.ds(zoff + i * 112, 112)], zs)
               for i in range(14)]
        for d in zds:
            d.wait()
        plsc.subcore_barrier()

        def group(g, carry):
            d1 = pltpu.async_copy(srcf.at[pl.ds(e0 + g * SGRC, SGRC)], sst_v,
                                  ss0)
            d2 = pltpu.async_copy(dstf.at[pl.ds(e0 + g * SGRC, SGRC)], dst_v,
                                  ss1)
            d1.wait()
            d2.wait()

            def scanvec(i, cnt):
                d = dst_v[pl.ds(i * 16, 16)]
                sv = sst_v[pl.ds(i * 16, 16)]
                m = (d >= lo) & (d < hi)
                plsc.store_compressed(cdst_v.at[pl.ds(cnt, 16)], d - lo,
                                      mask=m)
                plsc.store_compressed(csrc_v.at[pl.ds(cnt, 16)], sv, mask=m)
                return cnt + extract0(plsc.all_reduce_population_count(m))

            cnt = lax.fori_loop(0, SGRC // 16, scanvec, jnp.int32(0))

            for k in range(CHUNK // 16):
                csrc_v[pl.ds(cnt + k * 16, 16)] = jnp.zeros((16,), jnp.int32)
                cdst_v[pl.ds(cnt + k * 16, 16)] = jnp.full((16,), CSINK,
                                                           jnp.int32)
            nch = lax.shift_right_logical(cnt + (CHUNK - 1), 7)

            def stream(q, carry2):
                r = q & 1

                for rr in range(2):
                    @pl.when((q >= 2) & (r == rr))
                    def _():
                        drain(tp.at[pl.ds(0, CHUNK)], rows[rr], asem[rr])

                for rr in range(2):
                    @pl.when(r == rr)
                    def _():
                        for k in range(CHUNK // 16):
                            sidx[rr][pl.ds(k * 16, 16)] = (
                                csrc_v[pl.ds(q * CHUNK + k * 16, 16)])
                            didx[rr][pl.ds(k * 16, 16)] = (
                                cdst_v[pl.ds(q * CHUNK + k * 16, 16)])
                        pltpu.async_copy(tp.at[sidx[rr]], rows[rr], gsem[rr])

                @pl.when(q >= 1)
                def _():
                    for rr in range(2):
                        @pl.when(r == rr)
                        def _():
                            rp = 1 - rr
                            drain(tp.at[pl.ds(0, CHUNK)], rows[rp], gsem[rp])
                            pltpu.async_copy(rows[rp],
                                             buf_sh.at[didx[rp]],
                                             asem[rp], add=True)
                return carry2

            lax.fori_loop(0, nch, stream, 0)

            @pl.when(nch >= 1)
            def _():
                for rr in range(2):
                    @pl.when(((nch - 1) & 1) == rr)
                    def _():
                        drain(tp.at[pl.ds(0, CHUNK)], rows[rr], gsem[rr])
                        pltpu.async_copy(rows[rr], buf_sh.at[didx[rr]],
                                         asem[rr], add=True)
                        drain(tp.at[pl.ds(0, CHUNK)], rows[rr], asem[rr])

            @pl.when(nch >= 2)
            def _():
                for rr in range(2):
                    @pl.when(((nch - 1) & 1) != rr)
                    def _():
                        drain(tp.at[pl.ds(0, CHUNK)], rows[rr], asem[rr])
            return carry

        lax.fori_loop(0, ngr, group, 0)
        plsc.subcore_barrier()

        outs = [None, None]
        for i in range(14):
            r = i & 1
            if outs[r] is not None:
                outs[r].wait()
            pltpu.async_copy(buf_sh.at[pl.ds(ooff + i * 112, 112)],
                             rows[r].at[pl.ds(0, 112)], gsem[r]).wait()
            outs[r] = pltpu.async_copy(rows[r].at[pl.ds(0, 112)],
                                       op.at[pl.ds(lo + ooff + i * 112, 112)],
                                       osem[r])
        for d in outs:
            d.wait()
        plsc.subcore_barrier()


def _aggw_sc(tchunks, src_r, dst_r, zeros_c):
    return pl.kernel(
        _aggw_body,
        out_type=[jax.ShapeDtypeStruct((N, FW), jnp.float32)] * FK,
        mesh=plsc.VectorSubcoreMesh(core_axis_name="c", subcore_axis_name="s"),
        scratch_types=[
            pltpu.VMEM((SGRC,), jnp.int32),
            pltpu.VMEM((SGRC,), jnp.int32),
            pltpu.VMEM((CCAP + 16,), jnp.int32),
            pltpu.VMEM((CCAP + 16,), jnp.int32),
            pltpu.VMEM((CHUNK,), jnp.int32),
            pltpu.VMEM((CHUNK,), jnp.int32),
            pltpu.VMEM((CHUNK,), jnp.int32),
            pltpu.VMEM((CHUNK,), jnp.int32),
            pltpu.VMEM((CHUNK, FW), jnp.float32),
            pltpu.VMEM((CHUNK, FW), jnp.float32),
            pltpu.SemaphoreType.DMA,
            pltpu.SemaphoreType.DMA,
            pltpu.SemaphoreType.DMA,
            pltpu.SemaphoreType.DMA,
            pltpu.SemaphoreType.DMA,
            pltpu.SemaphoreType.DMA,
            pltpu.SemaphoreType.DMA,
            pltpu.SemaphoreType.DMA,
            pltpu.SemaphoreType.DMA,
            pltpu.VMEM_SHARED((CBUF, FW), jnp.float32),
        ],
        compiler_params=pltpu.CompilerParams(use_tc_tiling_on_sc=False, needs_layout_passes=False),
        interpret=_INTERPRET,
    )(*tchunks, src_r, dst_r, zeros_c)


def _aggn_body(t2t, srcr, dstr, zeros, out, src_v, dst_v, rows_v, zbuf_v,
               agg_sh):
    c = lax.axis_index("c")
    s = lax.axis_index("s")
    wid = c * NS + s
    pltpu.sync_copy(zeros.at[pl.ds(0, 391)], zbuf_v)
    for i in range(8):
        pltpu.sync_copy(zbuf_v, agg_sh.at[pl.ds(s * 3128 + i * 391, 391)])
    plsc.subcore_barrier()
    nrows = ROWS_ALL // (NC * NS)
    pltpu.sync_copy(srcr.at[pl.ds(wid * nrows, nrows)], src_v)
    pltpu.sync_copy(dstr.at[pl.ds(wid * nrows, nrows)], dst_v)

    def step(j, carry):
        pltpu.sync_copy(t2t.at[src_v.at[j]], rows_v)
        pltpu.sync_copy(rows_v, agg_sh.at[dst_v.at[j]], add=True)
        return carry

    lax.fori_loop(0, nrows, step, 0)
    plsc.subcore_barrier()
    for i in range(8):
        pltpu.sync_copy(agg_sh.at[pl.ds(s * 3128 + i * 391, 391)], zbuf_v)
        pltpu.sync_copy(zbuf_v, out.at[pl.ds(c * NP + s * 3128 + i * 391, 391)])


def _aggn_sc(t2, src_r, dst_r, zeros_e):
    nrows = ROWS_ALL // (NC * NS)
    return pl.kernel(
        _aggn_body,
        out_type=jax.ShapeDtypeStruct((NC * NP, 16), jnp.float32),
        mesh=plsc.VectorSubcoreMesh(core_axis_name="c", subcore_axis_name="s"),
        scratch_types=[
            pltpu.VMEM((nrows, CHUNK), jnp.int32),
            pltpu.VMEM((nrows, CHUNK), jnp.int32),
            pltpu.VMEM((CHUNK, 16), jnp.float32),
            pltpu.VMEM((391, 16), jnp.float32),
            pltpu.VMEM_SHARED((NP, 16), jnp.float32),
        ],
        compiler_params=pltpu.CompilerParams(use_tc_tiling_on_sc=False, needs_layout_passes=False),
        interpret=_INTERPRET,
    )(t2, src_r, dst_r, zeros_e)


def _phase_b_body(x_ref, w1t_ref, b1_ref, w1c0_ref, mask_ref, d0_ref, d1_ref,
                  *out_refs):
    xs = x_ref[...]
    x0 = xs[:, 0:1]
    rowsq = jnp.sum(xs * xs, axis=1, keepdims=True)
    yn2 = rowsq - x0 * x0
    theta = jnp.maximum(x0, MIN_T)
    scale = _acosh(theta) / jnp.sqrt(yn2 + 1e-15)
    mm = jnp.dot(xs, w1t_ref[...], preferred_element_type=jnp.float32)
    h1 = scale * (mm - x0 * w1c0_ref[...]) + b1_ref[...]
    t1 = h1 * mask_ref[...] * _dinv2(d0_ref[...], d1_ref[...])
    for p, oref in enumerate(out_refs):
        oref[...] = t1[:, p * FW:(p + 1) * FW]


def _phase_b(x, w1t, b1r, w1c0, mask, d0, d1):
    grid = (N // R_B,)
    return pl.pallas_call(
        _phase_b_body,
        grid=grid,
        in_specs=[
            pl.BlockSpec((R_B, D_IN), lambda i: (i, 0)),
            pl.BlockSpec((D_IN, HID), lambda i: (0, 0)),
            pl.BlockSpec((1, HID), lambda i: (0, 0)),
            pl.BlockSpec((1, HID), lambda i: (0, 0)),
            pl.BlockSpec((1, HID), lambda i: (0, 0)),
            pl.BlockSpec((R_B, 1), lambda i: (i, 0)),
            pl.BlockSpec((R_B, 1), lambda i: (i, 0)),
        ],
        out_specs=[pl.BlockSpec((R_B, FW), lambda i: (i, 0))] * FK,
        out_shape=[jax.ShapeDtypeStruct((N, FW), jnp.float32)] * FK,
        interpret=_INTERPRET,
    )(x, w1t, b1r, w1c0, mask, d0, d1)


def _phase_d_body(a0_ref, a1_ref, a2_ref, a3_ref, w2t_ref, b2_ref, mask_ref,
                  d0_ref, d1_ref, out_ref):
    agg = jnp.concatenate(
        [a0_ref[...], a1_ref[...], a2_ref[...], a3_ref[...]], axis=1)
    dinv = _dinv2(d0_ref[...], d1_ref[...])
    u2 = jax.nn.relu(agg * dinv)
    h2 = jnp.dot(u2, w2t_ref[...], preferred_element_type=jnp.float32)
    out_ref[...] = (h2 + b2_ref[...]) * mask_ref[...] * dinv


def _phase_d(aggs, w2t, b2r, mask, d0, d1):
    grid = (N // R_D,)
    return pl.pallas_call(
        _phase_d_body,
        grid=grid,
        in_specs=[pl.BlockSpec((R_D, FW), lambda i: (i, 0))] * FK + [
            pl.BlockSpec((HID, 16), lambda i: (0, 0)),
            pl.BlockSpec((1, 16), lambda i: (0, 0)),
            pl.BlockSpec((1, 16), lambda i: (0, 0)),
            pl.BlockSpec((R_D, 1), lambda i: (i, 0)),
            pl.BlockSpec((R_D, 1), lambda i: (i, 0)),
        ],
        out_specs=pl.BlockSpec((R_D, 16), lambda i: (i, 0)),
        out_shape=jax.ShapeDtypeStruct((N, 16), jnp.float32),
        interpret=_INTERPRET,
    )(*aggs, w2t, b2r, mask, d0, d1)


def _phase_f_body(p0_ref, p1_ref, d0_ref, d1_ref, out_ref):
    a = (p0_ref[...] + p1_ref[...]) * _dinv2(d0_ref[...], d1_ref[...])
    yn2 = jnp.sum(a * a, axis=1, keepdims=True)
    yn = jnp.sqrt(yn2 + 1e-15)
    s = 0.5 * (jnp.exp(yn) - jnp.exp(-yn)) / yn
    xr = a * s
    o0 = jnp.sqrt(1.0 + jnp.sum(xr * xr, axis=1, keepdims=True))
    out_ref[...] = jnp.concatenate([o0, xr[:, 1:D_OUT]], axis=1)


def _phase_f(p0, p1, d0, d1):
    grid = (N // R_F,)
    return pl.pallas_call(
        _phase_f_body,
        grid=grid,
        in_specs=[
            pl.BlockSpec((R_F, 16), lambda i: (i, 0)),
            pl.BlockSpec((R_F, 16), lambda i: (i, 0)),
            pl.BlockSpec((R_F, 1), lambda i: (i, 0)),
            pl.BlockSpec((R_F, 1), lambda i: (i, 0)),
        ],
        out_specs=pl.BlockSpec((R_F, D_OUT), lambda i: (i, 0)),
        out_shape=jax.ShapeDtypeStruct((N, D_OUT), jnp.float32),
        interpret=_INTERPRET,
    )(p0, p1, d0, d1)


def kernel(x, edge_index, W1, b1, W2, b2):
    src = edge_index[0].astype(jnp.int32)
    dst = edge_index[1].astype(jnp.int32)
    src_p = jnp.concatenate([src, jnp.zeros((E_PAD - E,), jnp.int32)])
    dst_p = jnp.concatenate([dst, jnp.full((E_PAD - E,), SINK, jnp.int32)])
    src_r = src_p.reshape(ROWS_ALL, CHUNK)
    dst_r = dst_p.reshape(ROWS_ALL, CHUNK)

    zeros_np = jnp.zeros((NP,), jnp.float32)
    zeros_c = jnp.zeros((CHUNK, FW), jnp.float32)
    zeros_e = jnp.zeros((NP, 16), jnp.float32)

    deg2 = _deg_sc(dst_r, zeros_np).reshape(NC, NP)
    d0 = deg2[0].reshape(NP, 1)
    d1 = deg2[1].reshape(NP, 1)

    w1t = W1.T
    b1r = b1.reshape(1, HID)
    w1c0 = W1[:, 0].reshape(1, HID)
    mask_h = jnp.ones((1, HID), jnp.float32).at[0, 0].set(0.0)

    tchunks = _phase_b(x, w1t, b1r, w1c0, mask_h, d0[:N], d1[:N])

    aggs = _aggw_sc(tchunks, src_p, dst_p, zeros_c)

    w2t = jnp.zeros((HID, 16), jnp.float32).at[:, :D_OUT].set(W2.T)
    b2r = jnp.zeros((1, 16), jnp.float32).at[0, :D_OUT].set(b2)
    mask16 = jnp.zeros((1, 16), jnp.float32).at[0, 1:D_OUT].set(1.0)

    t2 = _phase_d(aggs, w2t, b2r, mask16, d0[:N], d1[:N])

    agg2 = _aggn_sc(t2, src_r, dst_r, zeros_e).reshape(NC, NP, 16)

    return _phase_f(agg2[0], agg2[1], d0, d1)


# speedup vs baseline: 4.1926x; 1.0035x over previous
import functools

import jax
import jax.numpy as jnp
from jax import lax
from jax.experimental import pallas as pl
from jax.experimental.pallas import tpu as pltpu
from jax.experimental.pallas import tpu_sc as plsc

N = 50000
D_IN = 1433
HID = 256
D_OUT = 7
MIN_T = 1.0 + 1e-6
E = 800000

NC, NS = 2, 16
CHUNK = 128
E_PAD = 819200
ROWS_ALL = E_PAD // CHUNK
NP = 50048
SINK = 50040
HALF = 25000
CBUF = 25024
CSINK = 25000
FK = 4
FW = 64
FE = 8

_INTERPRET = False

R_B = 1000
R_D = 2000
R_F = 2000


def _acosh(t):
    return jnp.log(t + jnp.sqrt(t * t - 1.0))


def _dinv2(d0, d1):
    return lax.rsqrt(jnp.maximum(d0 + d1, 1.0))


def _deg_body(dstr, zeros, out, dst_v, ones_v, zbuf_v, deg_sh):
    c = lax.axis_index("c")
    s = lax.axis_index("s")
    wid = c * NS + s
    for i in range(CHUNK // 16):
        ones_v[pl.ds(i * 16, 16)] = jnp.ones((16,), jnp.float32)
    pltpu.sync_copy(zeros.at[pl.ds(0, 3128)], zbuf_v)
    pltpu.sync_copy(zbuf_v, deg_sh.at[pl.ds(s * 3128, 3128)])
    plsc.subcore_barrier()
    nrows = ROWS_ALL // (NC * NS)
    pltpu.sync_copy(dstr.at[pl.ds(wid * nrows, nrows)], dst_v)

    def step(j, carry):
        pltpu.sync_copy(ones_v, deg_sh.at[dst_v.at[j]], add=True)
        return carry

    lax.fori_loop(0, nrows, step, 0)
    plsc.subcore_barrier()
    pltpu.sync_copy(deg_sh.at[pl.ds(s * 3128, 3128)], zbuf_v)
    pltpu.sync_copy(zbuf_v, out.at[pl.ds(c * NP + s * 3128, 3128)])


def _deg_sc(dst_r, zeros_np):
    nrows = ROWS_ALL // (NC * NS)
    return pl.kernel(
        _deg_body,
        out_type=jax.ShapeDtypeStruct((NC * NP,), jnp.float32),
        mesh=plsc.VectorSubcoreMesh(core_axis_name="c", subcore_axis_name="s"),
        scratch_types=[
            pltpu.VMEM((nrows, CHUNK), jnp.int32),
            pltpu.VMEM((CHUNK,), jnp.float32),
            pltpu.VMEM((3128,), jnp.float32),
            pltpu.VMEM_SHARED((NP,), jnp.float32),
        ],
        compiler_params=pltpu.CompilerParams(use_tc_tiling_on_sc=False, needs_layout_passes=False),
        interpret=_INTERPRET,
    )(dst_r, zeros_np)


SGR = 16
CCAP = SGR * CHUNK + CHUNK


ROWW = CHUNK * FW


SGRC = SGR * CHUNK


def _aggw_body(t0, t1, t2, t3, srcf, dstf, zeros, o0, o1, o2, o3,
               sst_v, dst_v, csrc_v, cdst_v, sidx0, sidx1, didx0, didx1,
               rows0, rows1, gs0, gs1, as0, as1, ss0, ss1, zs, os0, os1,
               buf_sh):
    c = lax.axis_index("c")
    s = lax.axis_index("s")
    lo = c * HALF
    hi = lo + HALF
    nrows = ROWS_ALL // NS
    e0 = s * nrows * CHUNK
    ngr = nrows // SGR

    def extract0(v):
        return lax.squeeze(lax.slice(v, (0,), (1,)), (0,))

    def drain(dummy_hbm, buf, sem):
        pltpu.make_async_copy(dummy_hbm, buf, sem).wait()

    zrows = 1568
    zoff = jnp.minimum(s * zrows, CBUF - zrows)
    ooff = jnp.minimum(s * zrows, HALF - zrows)
    for tp, op in zip((t0, t1, t2, t3), (o0, o1, o2, o3)):
        rows = (rows0, rows1)
        sidx = (sidx0, sidx1)
        didx = (didx0, didx1)
        gsem = (gs0, gs1)
        asem = (as0, as1)
        osem = (os0, os1)

        pltpu.async_copy(zeros.at[pl.ds(0, CHUNK)], rows0, gs0).wait()
        zds = [pltpu.async_copy(rows0.at[pl.ds(0, 112)],
                                buf_sh.at[pl.ds(zoff + i * 112, 112)], zs)
               for i in range(14)]
        for d in zds:
            d.wait()
        plsc.subcore_barrier()

        def group(g, carry):
            d1 = pltpu.async_copy(srcf.at[pl.ds(e0 + g * SGRC, SGRC)], sst_v,
                                  ss0)
            d2 = pltpu.async_copy(dstf.at[pl.ds(e0 + g * SGRC, SGRC)], dst_v,
                                  ss1)
            d1.wait()
            d2.wait()

            def scanvec(i, cnt):
                d = dst_v[pl.ds(i * 16, 16)]
                sv = sst_v[pl.ds(i * 16, 16)]
                m = (d >= lo) & (d < hi)
                plsc.store_compressed(cdst_v.at[pl.ds(cnt, 16)], d - lo,
                                      mask=m)
                plsc.store_compressed(csrc_v.at[pl.ds(cnt, 16)], sv, mask=m)
                return cnt + extract0(plsc.all_reduce_population_count(m))

            cnt = lax.fori_loop(0, SGRC // 16, scanvec, jnp.int32(0))

            for k in range(CHUNK // 16):
                csrc_v[pl.ds(cnt + k * 16, 16)] = jnp.zeros((16,), jnp.int32)
                cdst_v[pl.ds(cnt + k * 16, 16)] = jnp.full((16,), CSINK,
                                                           jnp.int32)
            nch = lax.shift_right_logical(cnt + (CHUNK - 1), 7)

            def stream(q, carry2):
                r = q & 1

                for rr in range(2):
                    @pl.when((q >= 2) & (r == rr))
                    def _():
                        drain(tp.at[pl.ds(0, CHUNK)], rows[rr], asem[rr])

                for rr in range(2):
                    @pl.when(r == rr)
                    def _():
                        for k in range(CHUNK // 16):
                            sidx[rr][pl.ds(k * 16, 16)] = (
                                csrc_v[pl.ds(q * CHUNK + k * 16, 16)])
                            didx[rr][pl.ds(k * 16, 16)] = (
                                cdst_v[pl.ds(q * CHUNK + k * 16, 16)])
                        pltpu.async_copy(tp.at[sidx[rr]], rows[rr], gsem[rr])

                @pl.when(q >= 1)
                def _():
                    for rr in range(2):
                        @pl.when(r == rr)
                        def _():
                            rp = 1 - rr
                            drain(tp.at[pl.ds(0, CHUNK)], rows[rp], gsem[rp])
                            pltpu.async_copy(rows[rp],
                                             buf_sh.at[didx[rp]],
                                             asem[rp], add=True)
                return carry2

            lax.fori_loop(0, nch, stream, 0)

            @pl.when(nch >= 1)
            def _():
                for rr in range(2):
                    @pl.when(((nch - 1) & 1) == rr)
                    def _():
                        drain(tp.at[pl.ds(0, CHUNK)], rows[rr], gsem[rr])
                        pltpu.async_copy(rows[rr], buf_sh.at[didx[rr]],
                                         asem[rr], add=True)
                        drain(tp.at[pl.ds(0, CHUNK)], rows[rr], asem[rr])

            @pl.when(nch >= 2)
            def _():
                for rr in range(2):
                    @pl.when(((nch - 1) & 1) != rr)
                    def _():
                        drain(tp.at[pl.ds(0, CHUNK)], rows[rr], asem[rr])
            return carry

        lax.fori_loop(0, ngr, group, 0)
        plsc.subcore_barrier()

        outs = [None, None]
        for i in range(14):
            r = i & 1
            if outs[r] is not None:
                outs[r].wait()
            pltpu.async_copy(buf_sh.at[pl.ds(ooff + i * 112, 112)],
                             rows[r].at[pl.ds(0, 112)], gsem[r]).wait()
            outs[r] = pltpu.async_copy(rows[r].at[pl.ds(0, 112)],
                                       op.at[pl.ds(lo + ooff + i * 112, 112)],
                                       osem[r])
        for d in outs:
            d.wait()
        plsc.subcore_barrier()


def _aggw_sc(tchunks, src_r, dst_r, zeros_c):
    return pl.kernel(
        _aggw_body,
        out_type=[jax.ShapeDtypeStruct((N, FW), jnp.float32)] * FK,
        mesh=plsc.VectorSubcoreMesh(core_axis_name="c", subcore_axis_name="s"),
        scratch_types=[
            pltpu.VMEM((SGRC,), jnp.int32),
            pltpu.VMEM((SGRC,), jnp.int32),
            pltpu.VMEM((CCAP + 16,), jnp.int32),
            pltpu.VMEM((CCAP + 16,), jnp.int32),
            pltpu.VMEM((CHUNK,), jnp.int32),
            pltpu.VMEM((CHUNK,), jnp.int32),
            pltpu.VMEM((CHUNK,), jnp.int32),
            pltpu.VMEM((CHUNK,), jnp.int32),
            pltpu.VMEM((CHUNK, FW), jnp.float32),
            pltpu.VMEM((CHUNK, FW), jnp.float32),
            pltpu.SemaphoreType.DMA,
            pltpu.SemaphoreType.DMA,
            pltpu.SemaphoreType.DMA,
            pltpu.SemaphoreType.DMA,
            pltpu.SemaphoreType.DMA,
            pltpu.SemaphoreType.DMA,
            pltpu.SemaphoreType.DMA,
            pltpu.SemaphoreType.DMA,
            pltpu.SemaphoreType.DMA,
            pltpu.VMEM_SHARED((CBUF, FW), jnp.float32),
        ],
        compiler_params=pltpu.CompilerParams(use_tc_tiling_on_sc=False, needs_layout_passes=False),
        interpret=_INTERPRET,
    )(*tchunks, src_r, dst_r, zeros_c)


def _aggn_body(t2t, srcr, dstr, zeros, out, src_v, dst_v, rows_v, zbuf_v,
               agg_sh):
    c = lax.axis_index("c")
    s = lax.axis_index("s")
    wid = c * NS + s
    pltpu.sync_copy(zeros.at[pl.ds(0, 391)], zbuf_v)
    for i in range(8):
        pltpu.sync_copy(zbuf_v, agg_sh.at[pl.ds(s * 3128 + i * 391, 391)])
    plsc.subcore_barrier()
    nrows = ROWS_ALL // (NC * NS)
    pltpu.sync_copy(srcr.at[pl.ds(wid * nrows, nrows)], src_v)
    pltpu.sync_copy(dstr.at[pl.ds(wid * nrows, nrows)], dst_v)

    def step(j, carry):
        pltpu.sync_copy(t2t.at[src_v.at[j]], rows_v)
        pltpu.sync_copy(rows_v, agg_sh.at[dst_v.at[j]], add=True)
        return carry

    lax.fori_loop(0, nrows, step, 0)
    plsc.subcore_barrier()
    for i in range(8):
        pltpu.sync_copy(agg_sh.at[pl.ds(s * 3128 + i * 391, 391)], zbuf_v)
        pltpu.sync_copy(zbuf_v, out.at[pl.ds(c * NP + s * 3128 + i * 391, 391)])


def _aggn_sc(t2, src_r, dst_r, zeros_e):
    nrows = ROWS_ALL // (NC * NS)
    return pl.kernel(
        _aggn_body,
        out_type=jax.ShapeDtypeStruct((NC * NP, FE), jnp.float32),
        mesh=plsc.VectorSubcoreMesh(core_axis_name="c", subcore_axis_name="s"),
        scratch_types=[
            pltpu.VMEM((nrows, CHUNK), jnp.int32),
            pltpu.VMEM((nrows, CHUNK), jnp.int32),
            pltpu.VMEM((CHUNK, FE), jnp.float32),
            pltpu.VMEM((391, FE), jnp.float32),
            pltpu.VMEM_SHARED((NP, FE), jnp.float32),
        ],
        compiler_params=pltpu.CompilerParams(use_tc_tiling_on_sc=False, needs_layout_passes=False),
        interpret=_INTERPRET,
    )(t2, src_r, dst_r, zeros_e)


def _phase_b_body(x_ref, w1t_ref, b1_ref, w1c0_ref, mask_ref, d0_ref, d1_ref,
                  *out_refs):
    xs = x_ref[...]
    x0 = xs[:, 0:1]
    rowsq = jnp.sum(xs * xs, axis=1, keepdims=True)
    yn2 = rowsq - x0 * x0
    theta = jnp.maximum(x0, MIN_T)
    scale = _acosh(theta) / jnp.sqrt(yn2 + 1e-15)
    mm = jnp.dot(xs.astype(jnp.bfloat16), w1t_ref[...].astype(jnp.bfloat16),
                 preferred_element_type=jnp.float32)
    h1 = scale * (mm - x0 * w1c0_ref[...]) + b1_ref[...]
    t1 = h1 * mask_ref[...] * _dinv2(d0_ref[...], d1_ref[...])
    for p, oref in enumerate(out_refs):
        oref[...] = t1[:, p * FW:(p + 1) * FW]


def _phase_b(x, w1t, b1r, w1c0, mask, d0, d1):
    grid = (N // R_B,)
    return pl.pallas_call(
        _phase_b_body,
        grid=grid,
        in_specs=[
            pl.BlockSpec((R_B, D_IN), lambda i: (i, 0)),
            pl.BlockSpec((D_IN, HID), lambda i: (0, 0)),
            pl.BlockSpec((1, HID), lambda i: (0, 0)),
            pl.BlockSpec((1, HID), lambda i: (0, 0)),
            pl.BlockSpec((1, HID), lambda i: (0, 0)),
            pl.BlockSpec((R_B, 1), lambda i: (i, 0)),
            pl.BlockSpec((R_B, 1), lambda i: (i, 0)),
        ],
        out_specs=[pl.BlockSpec((R_B, FW), lambda i: (i, 0))] * FK,
        out_shape=[jax.ShapeDtypeStruct((N, FW), jnp.float32)] * FK,
        interpret=_INTERPRET,
    )(x, w1t, b1r, w1c0, mask, d0, d1)


def _phase_d_body(a0_ref, a1_ref, a2_ref, a3_ref, w2t_ref, b2_ref, mask_ref,
                  d0_ref, d1_ref, out_ref):
    agg = jnp.concatenate(
        [a0_ref[...], a1_ref[...], a2_ref[...], a3_ref[...]], axis=1)
    dinv = _dinv2(d0_ref[...], d1_ref[...])
    u2 = jax.nn.relu(agg * dinv)
    h2 = jnp.dot(u2, w2t_ref[...], preferred_element_type=jnp.float32)
    out_ref[...] = (h2 + b2_ref[...]) * mask_ref[...] * dinv


def _phase_d(aggs, w2t, b2r, mask, d0, d1):
    grid = (N // R_D,)
    return pl.pallas_call(
        _phase_d_body,
        grid=grid,
        in_specs=[pl.BlockSpec((R_D, FW), lambda i: (i, 0))] * FK + [
            pl.BlockSpec((HID, FE), lambda i: (0, 0)),
            pl.BlockSpec((1, FE), lambda i: (0, 0)),
            pl.BlockSpec((1, FE), lambda i: (0, 0)),
            pl.BlockSpec((R_D, 1), lambda i: (i, 0)),
            pl.BlockSpec((R_D, 1), lambda i: (i, 0)),
        ],
        out_specs=pl.BlockSpec((R_D, FE), lambda i: (i, 0)),
        out_shape=jax.ShapeDtypeStruct((N, FE), jnp.float32),
        interpret=_INTERPRET,
    )(*aggs, w2t, b2r, mask, d0, d1)


def _phase_f_body(p0_ref, p1_ref, d0_ref, d1_ref, out_ref):
    a = (p0_ref[...] + p1_ref[...]) * _dinv2(d0_ref[...], d1_ref[...])
    yn2 = jnp.sum(a * a, axis=1, keepdims=True)
    yn = jnp.sqrt(yn2 + 1e-15)
    s = 0.5 * (jnp.exp(yn) - jnp.exp(-yn)) / yn
    xr = a * s
    o0 = jnp.sqrt(1.0 + jnp.sum(xr * xr, axis=1, keepdims=True))
    out_ref[...] = jnp.concatenate([o0, xr[:, 1:D_OUT]], axis=1)


def _phase_f(p0, p1, d0, d1):
    grid = (N // R_F,)
    return pl.pallas_call(
        _phase_f_body,
        grid=grid,
        in_specs=[
            pl.BlockSpec((R_F, FE), lambda i: (i, 0)),
            pl.BlockSpec((R_F, FE), lambda i: (i, 0)),
            pl.BlockSpec((R_F, 1), lambda i: (i, 0)),
            pl.BlockSpec((R_F, 1), lambda i: (i, 0)),
        ],
        out_specs=pl.BlockSpec((R_F, D_OUT), lambda i: (i, 0)),
        out_shape=jax.ShapeDtypeStruct((N, D_OUT), jnp.float32),
        interpret=_INTERPRET,
    )(p0, p1, d0, d1)


def kernel(x, edge_index, W1, b1, W2, b2):
    src = edge_index[0].astype(jnp.int32)
    dst = edge_index[1].astype(jnp.int32)
    src_p = jnp.concatenate([src, jnp.zeros((E_PAD - E,), jnp.int32)])
    dst_p = jnp.concatenate([dst, jnp.full((E_PAD - E,), SINK, jnp.int32)])
    src_r = src_p.reshape(ROWS_ALL, CHUNK)
    dst_r = dst_p.reshape(ROWS_ALL, CHUNK)

    zeros_np = jnp.zeros((NP,), jnp.float32)
    zeros_c = jnp.zeros((CHUNK, FW), jnp.float32)
    zeros_e = jnp.zeros((NP, FE), jnp.float32)

    deg2 = _deg_sc(dst_r, zeros_np).reshape(NC, NP)
    d0 = deg2[0].reshape(NP, 1)
    d1 = deg2[1].reshape(NP, 1)

    w1t = W1.T
    b1r = b1.reshape(1, HID)
    w1c0 = W1[:, 0].reshape(1, HID)
    mask_h = jnp.ones((1, HID), jnp.float32).at[0, 0].set(0.0)

    tchunks = _phase_b(x, w1t, b1r, w1c0, mask_h, d0[:N], d1[:N])

    aggs = _aggw_sc(tchunks, src_p, dst_p, zeros_c)

    w2t = jnp.zeros((HID, FE), jnp.float32).at[:, :D_OUT].set(W2.T)
    b2r = jnp.zeros((1, FE), jnp.float32).at[0, :D_OUT].set(b2)
    mask16 = jnp.zeros((1, FE), jnp.float32).at[0, 1:D_OUT].set(1.0)

    t2 = _phase_d(aggs, w2t, b2r, mask16, d0[:N], d1[:N])

    agg2 = _aggn_sc(t2, src_r, dst_r, zeros_e).reshape(NC, NP, FE)

    return _phase_f(agg2[0], agg2[1], d0, d1)


# speedup vs baseline: 4.1938x; 1.0003x over previous
import jax
import jax.numpy as jnp
from jax import lax
from jax.experimental import pallas as pl
from jax.experimental.pallas import tpu as pltpu
from jax.experimental.pallas import tpu_sc as plsc

N = 50000
D_IN = 1433
HID = 256
D_OUT = 7
MIN_T = 1.0 + 1e-6
E = 800000

NC, NS = 2, 16
CHUNK = 128
E_PAD = 819200
ROWS_ALL = E_PAD // CHUNK
NP = 50048
SINK = 50040
HALF = 25000
CBUF = 25024
CSINK = 25000
FK = 4
FW = 64
FE = 8

R_B = 1000
R_D = 2000
R_F = 2000


def _acosh(t):
    return jnp.log(t + jnp.sqrt(t * t - 1.0))


def _dinv2(d0, d1):
    return lax.rsqrt(jnp.maximum(d0 + d1, 1.0))


def _deg_body(dstr, zeros, out, dst_v, ones_v, zbuf_v, deg_sh):
    c = lax.axis_index("c")
    s = lax.axis_index("s")
    wid = c * NS + s
    for i in range(CHUNK // 16):
        ones_v[pl.ds(i * 16, 16)] = jnp.ones((16,), jnp.float32)
    pltpu.sync_copy(zeros.at[pl.ds(0, 3128)], zbuf_v)
    pltpu.sync_copy(zbuf_v, deg_sh.at[pl.ds(s * 3128, 3128)])
    plsc.subcore_barrier()
    nrows = ROWS_ALL // (NC * NS)
    pltpu.sync_copy(dstr.at[pl.ds(wid * nrows, nrows)], dst_v)

    def step(j, carry):
        pltpu.sync_copy(ones_v, deg_sh.at[dst_v.at[j]], add=True)
        return carry

    lax.fori_loop(0, nrows, step, 0)
    plsc.subcore_barrier()
    pltpu.sync_copy(deg_sh.at[pl.ds(s * 3128, 3128)], zbuf_v)
    pltpu.sync_copy(zbuf_v, out.at[pl.ds(c * NP + s * 3128, 3128)])


def _deg_sc(dst_r, zeros_np):
    nrows = ROWS_ALL // (NC * NS)
    return pl.kernel(
        _deg_body,
        out_type=jax.ShapeDtypeStruct((NC * NP,), jnp.float32),
        mesh=plsc.VectorSubcoreMesh(core_axis_name="c", subcore_axis_name="s"),
        scratch_types=[
            pltpu.VMEM((nrows, CHUNK), jnp.int32),
            pltpu.VMEM((CHUNK,), jnp.float32),
            pltpu.VMEM((3128,), jnp.float32),
            pltpu.VMEM_SHARED((NP,), jnp.float32),
        ],
        compiler_params=pltpu.CompilerParams(use_tc_tiling_on_sc=False, needs_layout_passes=False),
    )(dst_r, zeros_np)


SGR = 16
CCAP = SGR * CHUNK + CHUNK


ROWW = CHUNK * FW


SGRC = SGR * CHUNK


def _aggw_body(t0, t1, t2, t3, srcf, dstf, zeros, o0, o1, o2, o3,
               sst_v, dst_v, csrc_v, cdst_v, sidx0, sidx1, didx0, didx1,
               rows0, rows1, gs0, gs1, as0, as1, ss0, ss1, zs, os0, os1,
               buf_sh):
    c = lax.axis_index("c")
    s = lax.axis_index("s")
    lo = c * HALF
    hi = lo + HALF
    nrows = ROWS_ALL // NS
    e0 = s * nrows * CHUNK
    ngr = nrows // SGR

    def extract0(v):
        return lax.squeeze(lax.slice(v, (0,), (1,)), (0,))

    def drain(dummy_hbm, buf, sem):
        pltpu.make_async_copy(dummy_hbm, buf, sem).wait()

    zrows = 1568
    zoff = jnp.minimum(s * zrows, CBUF - zrows)
    ooff = jnp.minimum(s * zrows, HALF - zrows)
    for tp, op in zip((t0, t1, t2, t3), (o0, o1, o2, o3)):
        rows = (rows0, rows1)
        sidx = (sidx0, sidx1)
        didx = (didx0, didx1)
        gsem = (gs0, gs1)
        asem = (as0, as1)
        osem = (os0, os1)

        pltpu.async_copy(zeros.at[pl.ds(0, CHUNK)], rows0, gs0).wait()
        zds = [pltpu.async_copy(rows0.at[pl.ds(0, 112)],
                                buf_sh.at[pl.ds(zoff + i * 112, 112)], zs)
               for i in range(14)]
        for d in zds:
            d.wait()
        plsc.subcore_barrier()

        def group(g, carry):
            d1 = pltpu.async_copy(srcf.at[pl.ds(e0 + g * SGRC, SGRC)], sst_v,
                                  ss0)
            d2 = pltpu.async_copy(dstf.at[pl.ds(e0 + g * SGRC, SGRC)], dst_v,
                                  ss1)
            d1.wait()
            d2.wait()

            def scanvec(i, cnt):
                d = dst_v[pl.ds(i * 16, 16)]
                sv = sst_v[pl.ds(i * 16, 16)]
                m = (d >= lo) & (d < hi)
                plsc.store_compressed(cdst_v.at[pl.ds(cnt, 16)], d - lo,
                                      mask=m)
                plsc.store_compressed(csrc_v.at[pl.ds(cnt, 16)], sv, mask=m)
                return cnt + extract0(plsc.all_reduce_population_count(m))

            cnt = lax.fori_loop(0, SGRC // 16, scanvec, jnp.int32(0))

            for k in range(CHUNK // 16):
                csrc_v[pl.ds(cnt + k * 16, 16)] = jnp.zeros((16,), jnp.int32)
                cdst_v[pl.ds(cnt + k * 16, 16)] = jnp.full((16,), CSINK,
                                                           jnp.int32)
            nch = lax.shift_right_logical(cnt + (CHUNK - 1), 7)

            def stream(q, carry2):
                r = q & 1

                for rr in range(2):
                    @pl.when((q >= 2) & (r == rr))
                    def _():
                        drain(tp.at[pl.ds(0, CHUNK)], rows[rr], asem[rr])

                for rr in range(2):
                    @pl.when(r == rr)
                    def _():
                        for k in range(CHUNK // 16):
                            sidx[rr][pl.ds(k * 16, 16)] = (
                                csrc_v[pl.ds(q * CHUNK + k * 16, 16)])
                            didx[rr][pl.ds(k * 16, 16)] = (
                                cdst_v[pl.ds(q * CHUNK + k * 16, 16)])
                        pltpu.async_copy(tp.at[sidx[rr]], rows[rr], gsem[rr])

                @pl.when(q >= 1)
                def _():
                    for rr in range(2):
                        @pl.when(r == rr)
                        def _():
                            rp = 1 - rr
                            drain(tp.at[pl.ds(0, CHUNK)], rows[rp], gsem[rp])
                            pltpu.async_copy(rows[rp],
                                             buf_sh.at[didx[rp]],
                                             asem[rp], add=True)
                return carry2

            lax.fori_loop(0, nch, stream, 0)

            @pl.when(nch >= 1)
            def _():
                for rr in range(2):
                    @pl.when(((nch - 1) & 1) == rr)
                    def _():
                        drain(tp.at[pl.ds(0, CHUNK)], rows[rr], gsem[rr])
                        pltpu.async_copy(rows[rr], buf_sh.at[didx[rr]],
                                         asem[rr], add=True)
                        drain(tp.at[pl.ds(0, CHUNK)], rows[rr], asem[rr])

            @pl.when(nch >= 2)
            def _():
                for rr in range(2):
                    @pl.when(((nch - 1) & 1) != rr)
                    def _():
                        drain(tp.at[pl.ds(0, CHUNK)], rows[rr], asem[rr])
            return carry

        lax.fori_loop(0, ngr, group, 0)
        plsc.subcore_barrier()

        outs = [None, None]
        for i in range(14):
            r = i & 1
            if outs[r] is not None:
                outs[r].wait()
            pltpu.async_copy(buf_sh.at[pl.ds(ooff + i * 112, 112)],
                             rows[r].at[pl.ds(0, 112)], gsem[r]).wait()
            outs[r] = pltpu.async_copy(rows[r].at[pl.ds(0, 112)],
                                       op.at[pl.ds(lo + ooff + i * 112, 112)],
                                       osem[r])
        for d in outs:
            d.wait()
        plsc.subcore_barrier()


def _aggw_sc(tchunks, src_r, dst_r, zeros_c):
    return pl.kernel(
        _aggw_body,
        out_type=[jax.ShapeDtypeStruct((N, FW), jnp.float32)] * FK,
        mesh=plsc.VectorSubcoreMesh(core_axis_name="c", subcore_axis_name="s"),
        scratch_types=[
            pltpu.VMEM((SGRC,), jnp.int32),
            pltpu.VMEM((SGRC,), jnp.int32),
            pltpu.VMEM((CCAP + 16,), jnp.int32),
            pltpu.VMEM((CCAP + 16,), jnp.int32),
            pltpu.VMEM((CHUNK,), jnp.int32),
            pltpu.VMEM((CHUNK,), jnp.int32),
            pltpu.VMEM((CHUNK,), jnp.int32),
            pltpu.VMEM((CHUNK,), jnp.int32),
            pltpu.VMEM((CHUNK, FW), jnp.float32),
            pltpu.VMEM((CHUNK, FW), jnp.float32),
            pltpu.SemaphoreType.DMA,
            pltpu.SemaphoreType.DMA,
            pltpu.SemaphoreType.DMA,
            pltpu.SemaphoreType.DMA,
            pltpu.SemaphoreType.DMA,
            pltpu.SemaphoreType.DMA,
            pltpu.SemaphoreType.DMA,
            pltpu.SemaphoreType.DMA,
            pltpu.SemaphoreType.DMA,
            pltpu.VMEM_SHARED((CBUF, FW), jnp.float32),
        ],
        compiler_params=pltpu.CompilerParams(use_tc_tiling_on_sc=False, needs_layout_passes=False),
    )(*tchunks, src_r, dst_r, zeros_c)


def _aggn_body(t2t, srcr, dstr, zeros, out, src_v, dst_v, rows_v, zbuf_v,
               agg_sh):
    c = lax.axis_index("c")
    s = lax.axis_index("s")
    wid = c * NS + s
    pltpu.sync_copy(zeros.at[pl.ds(0, 391)], zbuf_v)
    for i in range(8):
        pltpu.sync_copy(zbuf_v, agg_sh.at[pl.ds(s * 3128 + i * 391, 391)])
    plsc.subcore_barrier()
    nrows = ROWS_ALL // (NC * NS)
    pltpu.sync_copy(srcr.at[pl.ds(wid * nrows, nrows)], src_v)
    pltpu.sync_copy(dstr.at[pl.ds(wid * nrows, nrows)], dst_v)

    def step(j, carry):
        pltpu.sync_copy(t2t.at[src_v.at[j]], rows_v)
        pltpu.sync_copy(rows_v, agg_sh.at[dst_v.at[j]], add=True)
        return carry

    lax.fori_loop(0, nrows, step, 0)
    plsc.subcore_barrier()
    for i in range(8):
        pltpu.sync_copy(agg_sh.at[pl.ds(s * 3128 + i * 391, 391)], zbuf_v)
        pltpu.sync_copy(zbuf_v, out.at[pl.ds(c * NP + s * 3128 + i * 391, 391)])


def _aggn_sc(t2, src_r, dst_r, zeros_e):
    nrows = ROWS_ALL // (NC * NS)
    return pl.kernel(
        _aggn_body,
        out_type=jax.ShapeDtypeStruct((NC * NP, FE), jnp.float32),
        mesh=plsc.VectorSubcoreMesh(core_axis_name="c", subcore_axis_name="s"),
        scratch_types=[
            pltpu.VMEM((nrows, CHUNK), jnp.int32),
            pltpu.VMEM((nrows, CHUNK), jnp.int32),
            pltpu.VMEM((CHUNK, FE), jnp.float32),
            pltpu.VMEM((391, FE), jnp.float32),
            pltpu.VMEM_SHARED((NP, FE), jnp.float32),
        ],
        compiler_params=pltpu.CompilerParams(use_tc_tiling_on_sc=False, needs_layout_passes=False),
    )(t2, src_r, dst_r, zeros_e)


def _phase_b_body(x_ref, w1t_ref, b1_ref, w1c0_ref, mask_ref, d0_ref, d1_ref,
                  *out_refs):
    xs = x_ref[...]
    x0 = xs[:, 0:1]
    rowsq = jnp.sum(xs * xs, axis=1, keepdims=True)
    yn2 = rowsq - x0 * x0
    theta = jnp.maximum(x0, MIN_T)
    scale = _acosh(theta) / jnp.sqrt(yn2 + 1e-15)
    mm = jnp.dot(xs.astype(jnp.bfloat16), w1t_ref[...].astype(jnp.bfloat16),
                 preferred_element_type=jnp.float32)
    h1 = scale * (mm - x0 * w1c0_ref[...]) + b1_ref[...]
    t1 = h1 * mask_ref[...] * _dinv2(d0_ref[...], d1_ref[...])
    for p, oref in enumerate(out_refs):
        oref[...] = t1[:, p * FW:(p + 1) * FW]


def _phase_b(x, w1t, b1r, w1c0, mask, d0, d1):
    grid = (N // R_B,)
    return pl.pallas_call(
        _phase_b_body,
        grid=grid,
        in_specs=[
            pl.BlockSpec((R_B, D_IN), lambda i: (i, 0)),
            pl.BlockSpec((D_IN, HID), lambda i: (0, 0)),
            pl.BlockSpec((1, HID), lambda i: (0, 0)),
            pl.BlockSpec((1, HID), lambda i: (0, 0)),
            pl.BlockSpec((1, HID), lambda i: (0, 0)),
            pl.BlockSpec((R_B, 1), lambda i: (i, 0)),
            pl.BlockSpec((R_B, 1), lambda i: (i, 0)),
        ],
        out_specs=[pl.BlockSpec((R_B, FW), lambda i: (i, 0))] * FK,
        out_shape=[jax.ShapeDtypeStruct((N, FW), jnp.float32)] * FK,
    )(x, w1t, b1r, w1c0, mask, d0, d1)


def _phase_d_body(a0_ref, a1_ref, a2_ref, a3_ref, w2t_ref, b2_ref, mask_ref,
                  d0_ref, d1_ref, out_ref):
    agg = jnp.concatenate(
        [a0_ref[...], a1_ref[...], a2_ref[...], a3_ref[...]], axis=1)
    dinv = _dinv2(d0_ref[...], d1_ref[...])
    u2 = jax.nn.relu(agg * dinv)
    h2 = jnp.dot(u2, w2t_ref[...], preferred_element_type=jnp.float32)
    out_ref[...] = (h2 + b2_ref[...]) * mask_ref[...] * dinv


def _phase_d(aggs, w2t, b2r, mask, d0, d1):
    grid = (N // R_D,)
    return pl.pallas_call(
        _phase_d_body,
        grid=grid,
        in_specs=[pl.BlockSpec((R_D, FW), lambda i: (i, 0))] * FK + [
            pl.BlockSpec((HID, FE), lambda i: (0, 0)),
            pl.BlockSpec((1, FE), lambda i: (0, 0)),
            pl.BlockSpec((1, FE), lambda i: (0, 0)),
            pl.BlockSpec((R_D, 1), lambda i: (i, 0)),
            pl.BlockSpec((R_D, 1), lambda i: (i, 0)),
        ],
        out_specs=pl.BlockSpec((R_D, FE), lambda i: (i, 0)),
        out_shape=jax.ShapeDtypeStruct((N, FE), jnp.float32),
    )(*aggs, w2t, b2r, mask, d0, d1)


def _phase_f_body(p0_ref, p1_ref, d0_ref, d1_ref, out_ref):
    a = (p0_ref[...] + p1_ref[...]) * _dinv2(d0_ref[...], d1_ref[...])
    yn2 = jnp.sum(a * a, axis=1, keepdims=True)
    yn = jnp.sqrt(yn2 + 1e-15)
    s = 0.5 * (jnp.exp(yn) - jnp.exp(-yn)) / yn
    xr = a * s
    o0 = jnp.sqrt(1.0 + jnp.sum(xr * xr, axis=1, keepdims=True))
    out_ref[...] = jnp.concatenate([o0, xr[:, 1:D_OUT]], axis=1)


def _phase_f(p0, p1, d0, d1):
    grid = (N // R_F,)
    return pl.pallas_call(
        _phase_f_body,
        grid=grid,
        in_specs=[
            pl.BlockSpec((R_F, FE), lambda i: (i, 0)),
            pl.BlockSpec((R_F, FE), lambda i: (i, 0)),
            pl.BlockSpec((R_F, 1), lambda i: (i, 0)),
            pl.BlockSpec((R_F, 1), lambda i: (i, 0)),
        ],
        out_specs=pl.BlockSpec((R_F, D_OUT), lambda i: (i, 0)),
        out_shape=jax.ShapeDtypeStruct((N, D_OUT), jnp.float32),
    )(p0, p1, d0, d1)


def kernel(x, edge_index, W1, b1, W2, b2):
    src = edge_index[0].astype(jnp.int32)
    dst = edge_index[1].astype(jnp.int32)
    src_p = jnp.concatenate([src, jnp.zeros((E_PAD - E,), jnp.int32)])
    dst_p = jnp.concatenate([dst, jnp.full((E_PAD - E,), SINK, jnp.int32)])
    src_r = src_p.reshape(ROWS_ALL, CHUNK)
    dst_r = dst_p.reshape(ROWS_ALL, CHUNK)

    zeros_np = jnp.zeros((NP,), jnp.float32)
    zeros_c = jnp.zeros((CHUNK, FW), jnp.float32)
    zeros_e = jnp.zeros((NP, FE), jnp.float32)

    deg2 = _deg_sc(dst_r, zeros_np).reshape(NC, NP)
    d0 = deg2[0].reshape(NP, 1)
    d1 = deg2[1].reshape(NP, 1)

    w1t = W1.T
    b1r = b1.reshape(1, HID)
    w1c0 = W1[:, 0].reshape(1, HID)
    mask_h = jnp.ones((1, HID), jnp.float32).at[0, 0].set(0.0)

    tchunks = _phase_b(x, w1t, b1r, w1c0, mask_h, d0[:N], d1[:N])

    aggs = _aggw_sc(tchunks, src_p, dst_p, zeros_c)

    w2t = jnp.zeros((HID, FE), jnp.float32).at[:, :D_OUT].set(W2.T)
    b2r = jnp.zeros((1, FE), jnp.float32).at[0, :D_OUT].set(b2)
    mask16 = jnp.zeros((1, FE), jnp.float32).at[0, 1:D_OUT].set(1.0)

    t2 = _phase_d(aggs, w2t, b2r, mask16, d0[:N], d1[:N])

    agg2 = _aggn_sc(t2, src_r, dst_r, zeros_e).reshape(NC, NP, FE)

    return _phase_f(agg2[0], agg2[1], d0, d1)
